# baseline (device time: 90380 ns/iter reference)
import jax
import jax.numpy as jnp
from jax import lax
from jax.experimental import pallas as pl
from jax.experimental.pallas import tpu as pltpu

N_DEV = 4


def kernel(x, W1, W2):
    m, k = x.shape
    _, d = W1.shape
    _, f = W2.shape
    chunk = m // N_DEV

    def body(x_ref, w1_ref, w2_ref, out_ref,
             h_ref, rs_comm, ag_comm,
             rs_send, rs_recv, ag_send, ag_recv):
        my = lax.axis_index("i")
        left = lax.rem(my + N_DEV - 1, N_DEV)
        right = lax.rem(my + 1, N_DEV)

        barrier_sem = pltpu.get_barrier_semaphore()
        for nbr in (left, right):
            pl.semaphore_signal(
                barrier_sem, inc=1,
                device_id=(nbr,), device_id_type=pl.DeviceIdType.MESH,
            )
        pl.semaphore_wait(barrier_sem, 2)

        h_ref[...] = jnp.dot(x_ref[...], w1_ref[...],
                             preferred_element_type=jnp.float32)

        for s in range(N_DEV - 1):
            send_c = lax.rem(my - s + N_DEV, N_DEV)
            rdma = pltpu.make_async_remote_copy(
                src_ref=h_ref.at[pl.ds(send_c * chunk, chunk), :],
                dst_ref=rs_comm.at[s],
                send_sem=rs_send.at[s],
                recv_sem=rs_recv.at[s],
                device_id=(right,),
                device_id_type=pl.DeviceIdType.MESH,
            )
            rdma.start()
            rdma.wait()
            recv_c = lax.rem(my - 1 - s + N_DEV, N_DEV)
            h_ref[pl.ds(recv_c * chunk, chunk), :] = (
                h_ref[pl.ds(recv_c * chunk, chunk), :] + rs_comm[s]
            )

        own_c = lax.rem(my + 1, N_DEV)
        for t in range(N_DEV - 1):
            if t == 0:
                src = h_ref.at[pl.ds(own_c * chunk, chunk), :]
            else:
                src = ag_comm.at[t - 1]
            rdma = pltpu.make_async_remote_copy(
                src_ref=src,
                dst_ref=ag_comm.at[t],
                send_sem=ag_send.at[t],
                recv_sem=ag_recv.at[t],
                device_id=(right,),
                device_id_type=pl.DeviceIdType.MESH,
            )
            rdma.start()
            rdma.wait()
            got_c = lax.rem(my - t + N_DEV, N_DEV)
            h_ref[pl.ds(got_c * chunk, chunk), :] = ag_comm[t]

        out_ref[...] = jnp.dot(h_ref[...], w2_ref[...],
                               preferred_element_type=jnp.float32)

    return pl.pallas_call(
        body,
        out_shape=jax.ShapeDtypeStruct((m, f), jnp.float32),
        in_specs=[
            pl.BlockSpec(memory_space=pltpu.VMEM),
            pl.BlockSpec(memory_space=pltpu.VMEM),
            pl.BlockSpec(memory_space=pltpu.VMEM),
        ],
        out_specs=pl.BlockSpec(memory_space=pltpu.VMEM),
        scratch_shapes=[
            pltpu.VMEM((m, d), jnp.float32),
            pltpu.VMEM((N_DEV - 1, chunk, d), jnp.float32),
            pltpu.VMEM((N_DEV - 1, chunk, d), jnp.float32),
            pltpu.SemaphoreType.DMA((N_DEV - 1,)),
            pltpu.SemaphoreType.DMA((N_DEV - 1,)),
            pltpu.SemaphoreType.DMA((N_DEV - 1,)),
            pltpu.SemaphoreType.DMA((N_DEV - 1,)),
        ],
        compiler_params=pltpu.CompilerParams(collective_id=0),
    )(x, W1, W2)


# device time: 54970 ns/iter; 1.6442x vs baseline; 1.6442x over previous
import jax
import jax.numpy as jnp
from jax import lax
from jax.experimental import pallas as pl
from jax.experimental.pallas import tpu as pltpu

N_DEV = 4
R, L = 0, 1


def kernel(x, W1, W2):
    m, k = x.shape
    _, d = W1.shape
    _, f = W2.shape
    chunk = m // N_DEV
    d2 = d // 2

    def body(x_ref, w1_ref, w2_ref, out_ref,
             h_ref, rs_comm, ag_comm,
             rs_send, rs_recv, ag_send, ag_recv):
        my = lax.axis_index("i")
        left = lax.rem(my + N_DEV - 1, N_DEV)
        right = lax.rem(my + 1, N_DEV)

        barrier_sem = pltpu.get_barrier_semaphore()
        for nbr in (left, right):
            pl.semaphore_signal(
                barrier_sem, inc=1,
                device_id=(nbr,), device_id_type=pl.DeviceIdType.MESH,
            )
        pl.semaphore_wait(barrier_sem, 2)

        def compute_h(c):
            h_ref[pl.ds(c * chunk, chunk), :] = jnp.dot(
                x_ref[pl.ds(c * chunk, chunk), :], w1_ref[...],
                preferred_element_type=jnp.float32,
            )

        def compute_out(c):
            out_ref[pl.ds(c * chunk, chunk), :] = jnp.dot(
                h_ref[pl.ds(c * chunk, chunk), :], w2_ref[...],
                preferred_element_type=jnp.float32,
            )

        compute_h(my)

        for s in range(N_DEV - 1):
            send_r = lax.rem(my - s + N_DEV, N_DEV)
            send_l = lax.rem(my + s, N_DEV)
            rdma_r = pltpu.make_async_remote_copy(
                src_ref=h_ref.at[pl.ds(send_r * chunk, chunk), pl.ds(0, d2)],
                dst_ref=rs_comm.at[R, s],
                send_sem=rs_send.at[R, s],
                recv_sem=rs_recv.at[R, s],
                device_id=(right,),
                device_id_type=pl.DeviceIdType.MESH,
            )
            rdma_l = pltpu.make_async_remote_copy(
                src_ref=h_ref.at[pl.ds(send_l * chunk, chunk), pl.ds(d2, d2)],
                dst_ref=rs_comm.at[L, s],
                send_sem=rs_send.at[L, s],
                recv_sem=rs_recv.at[L, s],
                device_id=(left,),
                device_id_type=pl.DeviceIdType.MESH,
            )
            rdma_r.start()
            rdma_l.start()
            if s == 0:
                compute_h(lax.rem(my + 1, N_DEV))
                compute_h(lax.rem(my - 1 + N_DEV, N_DEV))
            elif s == 1:
                compute_h(lax.rem(my + 2, N_DEV))
            rdma_r.wait()
            rdma_l.wait()
            recv_r = lax.rem(my - s - 1 + N_DEV, N_DEV)
            recv_l = lax.rem(my + s + 1, N_DEV)
            h_ref[pl.ds(recv_r * chunk, chunk), pl.ds(0, d2)] = (
                h_ref[pl.ds(recv_r * chunk, chunk), pl.ds(0, d2)]
                + rs_comm[R, s]
            )
            h_ref[pl.ds(recv_l * chunk, chunk), pl.ds(d2, d2)] = (
                h_ref[pl.ds(recv_l * chunk, chunk), pl.ds(d2, d2)]
                + rs_comm[L, s]
            )

        own_r = lax.rem(my + 1, N_DEV)
        own_l = lax.rem(my - 1 + N_DEV, N_DEV)

        for t in range(N_DEV - 1):
            if t == 0:
                src_r = h_ref.at[pl.ds(own_r * chunk, chunk), pl.ds(0, d2)]
                src_l = h_ref.at[pl.ds(own_l * chunk, chunk), pl.ds(d2, d2)]
            else:
                src_r = ag_comm.at[R, t - 1]
                src_l = ag_comm.at[L, t - 1]
            rdma_r = pltpu.make_async_remote_copy(
                src_ref=src_r,
                dst_ref=ag_comm.at[R, t],
                send_sem=ag_send.at[R, t],
                recv_sem=ag_recv.at[R, t],
                device_id=(right,),
                device_id_type=pl.DeviceIdType.MESH,
            )
            rdma_l = pltpu.make_async_remote_copy(
                src_ref=src_l,
                dst_ref=ag_comm.at[L, t],
                send_sem=ag_send.at[L, t],
                recv_sem=ag_recv.at[L, t],
                device_id=(left,),
                device_id_type=pl.DeviceIdType.MESH,
            )
            rdma_r.start()
            rdma_l.start()
            if t == 1:
                compute_out(my)
            elif t == 2:
                compute_out(lax.rem(my + 1, N_DEV))
                compute_out(lax.rem(my - 1 + N_DEV, N_DEV))
            rdma_r.wait()
            rdma_l.wait()
            got_r = lax.rem(my - t + N_DEV, N_DEV)
            got_l = lax.rem(my + t, N_DEV)
            h_ref[pl.ds(got_r * chunk, chunk), pl.ds(0, d2)] = ag_comm[R, t]
            h_ref[pl.ds(got_l * chunk, chunk), pl.ds(d2, d2)] = ag_comm[L, t]

        compute_out(lax.rem(my + 2, N_DEV))

    return pl.pallas_call(
        body,
        out_shape=jax.ShapeDtypeStruct((m, f), jnp.float32),
        in_specs=[
            pl.BlockSpec(memory_space=pltpu.VMEM),
            pl.BlockSpec(memory_space=pltpu.VMEM),
            pl.BlockSpec(memory_space=pltpu.VMEM),
        ],
        out_specs=pl.BlockSpec(memory_space=pltpu.VMEM),
        scratch_shapes=[
            pltpu.VMEM((m, d), jnp.float32),
            pltpu.VMEM((2, N_DEV - 1, chunk, d // 2), jnp.float32),
            pltpu.VMEM((2, N_DEV - 1, chunk, d // 2), jnp.float32),
            pltpu.SemaphoreType.DMA((2, N_DEV - 1)),
            pltpu.SemaphoreType.DMA((2, N_DEV - 1)),
            pltpu.SemaphoreType.DMA((2, N_DEV - 1)),
            pltpu.SemaphoreType.DMA((2, N_DEV - 1)),
        ],
        compiler_params=pltpu.CompilerParams(collective_id=0),
    )(x, W1, W2)


# device time: 46667 ns/iter; 1.9367x vs baseline; 1.1779x over previous
import jax
import jax.numpy as jnp
from jax import lax
from jax.experimental import pallas as pl
from jax.experimental.pallas import tpu as pltpu

N_DEV = 4
R, L = 0, 1
S = 2


def kernel(x, W1, W2):
    m, k = x.shape
    _, d = W1.shape
    _, f = W2.shape
    chunk = m // N_DEV
    d2 = d // 2
    w = d2 // S

    def body(x_ref, w1_ref, w2_ref, out_ref,
             h_ref, rs_comm, ag_comm,
             rs_send, rs_recv, ag_send, ag_recv):
        my = lax.axis_index("i")
        left = lax.rem(my + N_DEV - 1, N_DEV)
        right = lax.rem(my + 1, N_DEV)

        def mod(e):
            return lax.rem(e + N_DEV, N_DEV)

        def rows(c):
            return pl.ds(c * chunk, chunk)

        def cols(d_, j):
            return pl.ds((0 if d_ == R else d2) + j * w, w)

        barrier_sem = pltpu.get_barrier_semaphore()
        for nbr in (left, right):
            pl.semaphore_signal(
                barrier_sem, inc=1,
                device_id=(nbr,), device_id_type=pl.DeviceIdType.MESH,
            )
        pl.semaphore_wait(barrier_sem, 2)

        def compute_h(c):
            h_ref[rows(c), :] = jnp.dot(
                x_ref[rows(c), :], w1_ref[...],
                preferred_element_type=jnp.float32,
            )

        def compute_out(c):
            out_ref[rows(c), :] = jnp.dot(
                h_ref[rows(c), :], w2_ref[...],
                preferred_element_type=jnp.float32,
            )

        def make_rs(d_, s, j, src_c):
            return pltpu.make_async_remote_copy(
                src_ref=h_ref.at[rows(src_c), cols(d_, j)],
                dst_ref=rs_comm.at[d_, s, j],
                send_sem=rs_send.at[d_, s, j],
                recv_sem=rs_recv.at[d_, s, j],
                device_id=(right if d_ == R else left,),
                device_id_type=pl.DeviceIdType.MESH,
            )

        def make_ag(d_, t, j, src):
            return pltpu.make_async_remote_copy(
                src_ref=src,
                dst_ref=ag_comm.at[d_, t, j],
                send_sem=ag_send.at[d_, t, j],
                recv_sem=ag_recv.at[d_, t, j],
                device_id=(right if d_ == R else left,),
                device_id_type=pl.DeviceIdType.MESH,
            )

        compute_h(my)
        inflight = {}
        for j in range(S):
            for d_ in (R, L):
                rd = make_rs(d_, 0, j, my)
                rd.start()
                inflight[(d_, 0, j)] = rd
        compute_h(mod(my + 1))
        compute_h(mod(my - 1))

        for s in range(1, N_DEV - 1):
            recv_r = mod(my - s)
            recv_l = mod(my + s)
            for j in range(S):
                for d_, rc in ((R, recv_r), (L, recv_l)):
                    inflight[(d_, s - 1, j)].wait()
                    h_ref[rows(rc), cols(d_, j)] = (
                        h_ref[rows(rc), cols(d_, j)] + rs_comm[d_, s - 1, j]
                    )
                    rd = make_rs(d_, s, j, rc)
                    rd.start()
                    inflight[(d_, s, j)] = rd
            if s == 1:
                compute_h(mod(my + 2))

        red_r = mod(my + 1)
        red_l = mod(my - 1)
        ag_inflight = {}
        for j in range(S):
            for d_, rc in ((R, red_r), (L, red_l)):
                inflight[(d_, N_DEV - 2, j)].wait()
                h_ref[rows(rc), cols(d_, j)] = (
                    h_ref[rows(rc), cols(d_, j)] + rs_comm[d_, N_DEV - 2, j]
                )
                rd = make_ag(d_, 0, j, h_ref.at[rows(rc), cols(d_, j)])
                rd.start()
                ag_inflight[(d_, 0, j)] = rd

        for t in range(1, N_DEV - 1):
            got_r = mod(my - (t - 1))
            got_l = mod(my + (t - 1))
            for j in range(S):
                for d_, gc in ((R, got_r), (L, got_l)):
                    ag_inflight[(d_, t - 1, j)].wait()
                    rd = make_ag(d_, t, j, ag_comm.at[d_, t - 1, j])
                    rd.start()
                    ag_inflight[(d_, t, j)] = rd
                    h_ref[rows(gc), cols(d_, j)] = ag_comm[d_, t - 1, j]
            if t == 1:
                compute_out(my)
            else:
                compute_out(mod(my + 1))
                compute_out(mod(my - 1))

        got_r = mod(my - 2)
        got_l = mod(my + 2)
        for j in range(S):
            for d_, gc in ((R, got_r), (L, got_l)):
                ag_inflight[(d_, N_DEV - 2, j)].wait()
                h_ref[rows(gc), cols(d_, j)] = ag_comm[d_, N_DEV - 2, j]
        compute_out(mod(my + 2))

    return pl.pallas_call(
        body,
        out_shape=jax.ShapeDtypeStruct((m, f), jnp.float32),
        in_specs=[
            pl.BlockSpec(memory_space=pltpu.VMEM),
            pl.BlockSpec(memory_space=pltpu.VMEM),
            pl.BlockSpec(memory_space=pltpu.VMEM),
        ],
        out_specs=pl.BlockSpec(memory_space=pltpu.VMEM),
        scratch_shapes=[
            pltpu.VMEM((m, d), jnp.float32),
            pltpu.VMEM((2, N_DEV - 1, S, chunk, d // 2 // S), jnp.float32),
            pltpu.VMEM((2, N_DEV - 1, S, chunk, d // 2 // S), jnp.float32),
            pltpu.SemaphoreType.DMA((2, N_DEV - 1, S)),
            pltpu.SemaphoreType.DMA((2, N_DEV - 1, S)),
            pltpu.SemaphoreType.DMA((2, N_DEV - 1, S)),
            pltpu.SemaphoreType.DMA((2, N_DEV - 1, S)),
        ],
        compiler_params=pltpu.CompilerParams(collective_id=0),
    )(x, W1, W2)
